# Initial kernel scaffold; baseline (speedup 1.0000x reference)
import jax
import jax.numpy as jnp
from jax.experimental import pallas as pl


def kernel(x, edge_index, edge_weight, W0, b0, W1, b1):
    raise NotImplementedError("write your pallas kernel here")


# speedup vs baseline: 4.0130x; 4.0130x over previous
import functools

import jax
import jax.numpy as jnp
from jax import lax
from jax.experimental import pallas as pl
from jax.experimental.pallas import tpu as pltpu
from jax.experimental.pallas import tpu_sc as plsc

N_NODES = 10000
N_PAD = 10240
DIM = 128
N_EDGES = 320000

NC = 2
NS = 16
NW = NC * NS

GPW = 79
EPW = GPW * 128
E_PAD = NW * EPW

ROWS_PER_TILE = N_PAD // NS


def _spmm_sc(src3d, dst3d, w3d, h):
    mesh = plsc.VectorSubcoreMesh(core_axis_name="c", subcore_axis_name="s")

    @functools.partial(
        pl.kernel,
        mesh=mesh,
        compiler_params=pltpu.CompilerParams(needs_layout_passes=False),
        out_type=jax.ShapeDtypeStruct((NC, N_PAD, DIM), jnp.float32),
        scratch_types=[
            pltpu.VMEM((GPW, 128), jnp.int32),
            pltpu.VMEM((GPW, 128), jnp.int32),
            pltpu.VMEM((GPW, 128), jnp.float32),
            pltpu.VMEM((128, DIM), jnp.float32),
            pltpu.VMEM_SHARED((N_PAD, DIM), jnp.float32),
            pltpu.SemaphoreType.DMA,
        ],
    )
    def spmm(src_hbm, dst_hbm, w_hbm, h_hbm, out_hbm,
             src_v, dst_v, w_v, rows_v, acc_sh, sem):
        c = lax.axis_index("c")
        s = lax.axis_index("s")
        wid = s * NC + c
        zvec = jnp.zeros((16,), jnp.float32)

        def zbody(r, _):
            for k in range(8):
                rows_v[r, pl.ds(k * 16, 16)] = zvec
            return 0
        lax.fori_loop(0, 128, zbody, 0)
        rbase = s * ROWS_PER_TILE
        for j in range(5):
            pltpu.sync_copy(rows_v, acc_sh.at[pl.ds(rbase + j * 128, 128)])
        plsc.subcore_barrier()

        pltpu.sync_copy(src_hbm.at[wid], src_v)
        pltpu.sync_copy(dst_hbm.at[wid], dst_v)
        pltpu.sync_copy(w_hbm.at[wid], w_v)

        def body(t, _):
            pltpu.async_copy(h_hbm.at[src_v.at[t]], rows_v, sem).wait()

            def rbody(r, _):
                wvec = plsc.load_gather(
                    w_v, [jnp.full((16,), t, jnp.int32),
                          jnp.full((16,), r, jnp.int32)])
                for k in range(8):
                    sl = pl.ds(k * 16, 16)
                    rows_v[r, sl] = rows_v[r, sl] * wvec
                return 0
            lax.fori_loop(0, 128, rbody, 0)

            pltpu.sync_copy(rows_v, acc_sh.at[dst_v.at[t]], add=True)
            return 0
        lax.fori_loop(0, GPW, body, 0)

        plsc.subcore_barrier()
        pltpu.sync_copy(acc_sh.at[pl.ds(rbase, ROWS_PER_TILE)],
                        out_hbm.at[c, pl.ds(rbase, ROWS_PER_TILE)])

    return spmm(src3d, dst3d, w3d, h)


_BLK = 1024


def _mm_bias(x, W, b):
    def body(x_ref, w_ref, b_ref, o_ref):
        o_ref[...] = (jnp.dot(x_ref[...], w_ref[...],
                              preferred_element_type=jnp.float32)
                      + b_ref[...])
    return pl.pallas_call(
        body,
        grid=(N_PAD // _BLK,),
        in_specs=[pl.BlockSpec((_BLK, DIM), lambda i: (i, 0)),
                  pl.BlockSpec((DIM, DIM), lambda i: (0, 0)),
                  pl.BlockSpec((1, DIM), lambda i: (0, 0))],
        out_specs=pl.BlockSpec((_BLK, DIM), lambda i: (i, 0)),
        out_shape=jax.ShapeDtypeStruct((N_PAD, DIM), jnp.float32),
    )(x, W, b.reshape(1, DIM))


def _add2(p):
    def body(p_ref, o_ref):
        o_ref[...] = p_ref[0] + p_ref[1]
    return pl.pallas_call(
        body,
        grid=(N_PAD // _BLK,),
        in_specs=[pl.BlockSpec((2, _BLK, DIM), lambda i: (0, i, 0))],
        out_specs=pl.BlockSpec((_BLK, DIM), lambda i: (i, 0)),
        out_shape=jax.ShapeDtypeStruct((N_PAD, DIM), jnp.float32),
    )(p)


def _final(p, W, b):
    def body(p_ref, w_ref, b_ref, o_ref):
        h = jnp.maximum(p_ref[0] + p_ref[1], 0.0)
        o_ref[...] = (jnp.dot(h, w_ref[...],
                              preferred_element_type=jnp.float32)
                      + b_ref[...])
    return pl.pallas_call(
        body,
        grid=(N_PAD // _BLK,),
        in_specs=[pl.BlockSpec((2, _BLK, DIM), lambda i: (0, i, 0)),
                  pl.BlockSpec((DIM, DIM), lambda i: (0, 0)),
                  pl.BlockSpec((1, DIM), lambda i: (0, 0))],
        out_specs=pl.BlockSpec((_BLK, DIM), lambda i: (i, 0)),
        out_shape=jax.ShapeDtypeStruct((N_PAD, DIM), jnp.float32),
    )(p, W, b.reshape(1, DIM))


def kernel(x, edge_index, edge_weight, W0, b0, W1, b1):
    pad = E_PAD - N_EDGES
    src = jnp.concatenate(
        [edge_index[0].astype(jnp.int32), jnp.zeros((pad,), jnp.int32)]
    ).reshape(NW, GPW, 128)
    dst = jnp.concatenate(
        [edge_index[1].astype(jnp.int32), jnp.zeros((pad,), jnp.int32)]
    ).reshape(NW, GPW, 128)
    w = jnp.concatenate(
        [edge_weight.astype(jnp.float32), jnp.zeros((pad,), jnp.float32)]
    ).reshape(NW, GPW, 128)

    x_p = jnp.pad(x, ((0, N_PAD - N_NODES), (0, 0)))
    h0 = _mm_bias(x_p, W0, b0)
    p1 = _spmm_sc(src, dst, w, h0)
    h1 = _add2(p1)
    p2 = _spmm_sc(src, dst, w, h1)
    out = _final(p2, W1, b1)
    return out[:N_NODES]
